# baseline (speedup 1.0000x reference)
import functools

import jax
import jax.numpy as jnp
from jax import lax
from jax.experimental import pallas as pl
from jax.experimental.pallas import tpu as pltpu
from jax.experimental.pallas import tpu_sc as plsc

_D = 64
_SCALE = 8.0
_NW = 32
_C = 128
_LANES = 16
_NG = 4
_NS = 2


@functools.partial(jax.jit, static_argnames=("nch",))
def _emb_lookup(tok, table, nch):
    mesh = plsc.VectorSubcoreMesh(core_axis_name="c", subcore_axis_name="s")

    @functools.partial(
        pl.kernel,
        mesh=mesh,
        out_type=jax.ShapeDtypeStruct((_NW, nch, _C, _D), jnp.float32),
        compiler_params=pltpu.CompilerParams(use_tc_tiling_on_sc=False),
        scratch_types=[
            pltpu.VMEM((nch, _C), jnp.int32),
        ]
        + [pltpu.VMEM((_C, _D), jnp.float32) for _ in range(_NG + _NS)]
        + [pltpu.SemaphoreType.DMA for _ in range(_NG + _NS)],
    )
    def body(tok_hbm, table_hbm, out_hbm, idx_v, *bufs_and_sems):
        gbufs = bufs_and_sems[:_NG]
        sbufs = bufs_and_sems[_NG:_NG + _NS]
        gsems = bufs_and_sems[_NG + _NS:2 * _NG + _NS]
        ssems = bufs_and_sems[2 * _NG + _NS:]

        cid = lax.axis_index("c")
        sid = lax.axis_index("s")
        wid = sid * 2 + cid

        pltpu.sync_copy(tok_hbm.at[wid], idx_v)

        def start_gather(chunk, g):
            pltpu.make_async_copy(
                table_hbm.at[idx_v.at[chunk]], gbufs[g], gsems[g]
            ).start()

        def wait_gather(chunk, g):
            pltpu.make_async_copy(
                table_hbm.at[idx_v.at[chunk]], gbufs[g], gsems[g]
            ).wait()

        def start_store(chunk, s):
            pltpu.make_async_copy(
                sbufs[s], out_hbm.at[wid, chunk], ssems[s]
            ).start()

        def wait_store(chunk, s):
            pltpu.make_async_copy(
                sbufs[s], out_hbm.at[wid, chunk], ssems[s]
            ).wait()

        n_vec = _C * _D // _LANES

        def scale(g, s):
            def scale_body(k, carry):
                r = k >> 2
                col = (k & 3) * _LANES
                sbufs[s][r, pl.ds(col, _LANES)] = (
                    gbufs[g][r, pl.ds(col, _LANES)] * _SCALE
                )
                return carry

            lax.fori_loop(0, n_vec, scale_body, 0, unroll=8)

        for g in range(_NG):
            start_gather(g, g)

        for g in range(_NG):
            chunk = g
            s = chunk % _NS
            wait_gather(chunk, g)
            if chunk >= _NS:
                wait_store(chunk - _NS, s)
            scale(g, s)
            start_store(chunk, s)
            start_gather(chunk + _NG, g)

        def loop_body(i, carry):
            for g in range(_NG):
                chunk = i * _NG + g
                s = g % _NS
                wait_gather(chunk, g)
                wait_store(chunk - _NS, s)
                scale(g, s)
                start_store(chunk, s)

                @pl.when(chunk + _NG < nch)
                def _():
                    start_gather(chunk + _NG, g)
            return carry

        lax.fori_loop(1, nch // _NG, loop_body, 0)

        for k in range(_NS):
            chunk = nch - _NS + k
            wait_store(chunk, chunk % _NS)

    return body(tok, table)


def kernel(tokens, embedding):
    bsz, seq = tokens.shape
    tot = bsz * seq
    nch = tot // (_NW * _C)
    tok = tokens.astype(jnp.int32).reshape(_NW, nch, _C)
    out = _emb_lookup(tok, embedding, nch)
    return out.reshape(bsz, seq, _D)
